# baseline (speedup 1.0000x reference)
import functools

import jax
import jax.numpy as jnp
from jax import lax
from jax.experimental import pallas as pl
from jax.experimental.pallas import tpu as pltpu
from jax.experimental.pallas import tpu_sc as plsc

_K = 8192
_D = 32
_B = 64
_T = 1024
_N = _B * _T

_info = plsc.get_sparse_core_info()
_NC, _NS = _info.num_cores, _info.num_subcores
_NW = _NC * _NS
_N_PER_W = _N // _NW
_NCH = 8
_CHUNK = _N_PER_W // _NCH


@functools.partial(
    pl.kernel,
    mesh=plsc.VectorSubcoreMesh(core_axis_name="c", subcore_axis_name="s"),
    out_type=jax.ShapeDtypeStruct((_N, _D), jnp.float32),
    scratch_types=[
        pltpu.VMEM((_N_PER_W,), jnp.int32),
        pltpu.VMEM((_N_PER_W, _D), jnp.float32),
        pltpu.SemaphoreType.DMA,
        pltpu.SemaphoreType.DMA,
        pltpu.SemaphoreType.DMA,
    ],
    compiler_params=pltpu.CompilerParams(use_tc_tiling_on_sc=False),
)
def _gather_rows(idx_hbm, table_hbm, out_hbm, idx_v, rows_v, gsem0, gsem1, wsem):
    wid = lax.axis_index("s") * _NC + lax.axis_index("c")
    base = wid * _N_PER_W
    gsems = (gsem0, gsem1)
    pltpu.sync_copy(idx_hbm.at[pl.ds(base, _N_PER_W)], idx_v)

    def start_gather(i):
        return pltpu.async_copy(
            table_hbm.at[idx_v.at[pl.ds(i * _CHUNK, _CHUNK)]],
            rows_v.at[pl.ds(i * _CHUNK, _CHUNK)],
            gsems[i % 2],
        )

    gds = [start_gather(0), start_gather(1)]
    wds = []
    for i in range(_NCH):
        gds[i].wait()
        wds.append(
            pltpu.async_copy(
                rows_v.at[pl.ds(i * _CHUNK, _CHUNK)],
                out_hbm.at[pl.ds(base + i * _CHUNK, _CHUNK)],
                wsem,
            )
        )
        if i + 2 < _NCH:
            gds.append(start_gather(i + 2))
    for d in wds:
        d.wait()


@jax.jit
def kernel(embed_id, weight):
    flat_ids = embed_id.reshape(_N)
    out = _gather_rows(flat_ids, weight)
    return out.reshape(_B, _T, _D)


# speedup vs baseline: 1.0168x; 1.0168x over previous
import functools

import jax
import jax.numpy as jnp
from jax import lax
from jax.experimental import pallas as pl
from jax.experimental.pallas import tpu as pltpu
from jax.experimental.pallas import tpu_sc as plsc

_K = 8192
_D = 32
_B = 64
_T = 1024
_N = _B * _T

_info = plsc.get_sparse_core_info()
_NC, _NS = _info.num_cores, _info.num_subcores
_NW = _NC * _NS
_ROWS_PER_W = _B // _NW
_N_PER_W = _ROWS_PER_W * _T


@functools.partial(
    pl.kernel,
    mesh=plsc.VectorSubcoreMesh(core_axis_name="c", subcore_axis_name="s"),
    out_type=jax.ShapeDtypeStruct((_B, _T, _D), jnp.float32),
    scratch_types=[
        pltpu.VMEM((_N_PER_W,), jnp.int32),
        pltpu.VMEM((_N_PER_W, _D), jnp.float32),
        pltpu.SemaphoreType.DMA,
    ],
    compiler_params=pltpu.CompilerParams(use_tc_tiling_on_sc=False),
)
def _gather_rows(idx_hbm, table_hbm, out_hbm, idx_v, rows_v, sem):
    wid = lax.axis_index("s") * _NC + lax.axis_index("c")
    row0 = wid * _ROWS_PER_W
    for r in range(_ROWS_PER_W):
        pltpu.sync_copy(idx_hbm.at[row0 + r], idx_v.at[pl.ds(r * _T, _T)])
    pltpu.async_copy(table_hbm.at[idx_v], rows_v, sem).wait()
    for r in range(_ROWS_PER_W):
        pltpu.sync_copy(rows_v.at[pl.ds(r * _T, _T)], out_hbm.at[row0 + r])


@jax.jit
def kernel(embed_id, weight):
    return _gather_rows(embed_id, weight)
